# baseline (speedup 1.0000x reference)
import functools

import jax
import jax.numpy as jnp
from jax.experimental import pallas as pl

B = 64
HW = 224
C_IN = 3
C_CONV = 64
N_CLS = 8
R_OUT = 8

QS = 57
ROWS = 226
MROWS = HW * QS
XROWS = ROWS * QS
NCHUNK = 4
CH = MROWS // NCHUNK
PIX = HW * HW


def _conv_gap_kernel(x_ref, w_ref, b_ref, o_ref):
    total = jnp.zeros((1, 256), dtype=jnp.float32)
    for c in range(NCHUNK):
        m0 = c * CH
        pf = jnp.concatenate(
            [x_ref[0, m0 + 57 * dy: m0 + 57 * dy + CH, :] for dy in range(3)],
            axis=1)
        ph = pf.astype(jnp.bfloat16)
        pl_ = (pf - ph.astype(jnp.float32)).astype(jnp.bfloat16)
        p3 = jnp.concatenate([ph, ph, pl_], axis=1)
        y = jax.lax.dot_general(
            p3, w_ref[...],
            dimension_numbers=(((1,), (0,)), ((), ())),
            preferred_element_type=jnp.float32)
        y = jnp.maximum(y + b_ref[0:1, :], 0.0)
        m_idx = jax.lax.broadcasted_iota(jnp.int32, (CH, 256), 0) + m0
        y = jnp.where((m_idx % QS) != (QS - 1), y, 0.0)
        total = total + jnp.sum(y, axis=0, keepdims=True)
    o_ref[0, 0, :] = total[0, :]


def _head_kernel(f_ref, wc_ref, bc_ref, wr_ref, br_ref, p_ref, r_ref):
    f = (f_ref[:, 0:64] + f_ref[:, 64:128] + f_ref[:, 128:192]
         + f_ref[:, 192:256]) * (1.0 / PIX)
    logits = jax.lax.dot_general(
        f, wc_ref[...], dimension_numbers=(((1,), (0,)), ((), ())),
        preferred_element_type=jnp.float32) + bc_ref[0:1, :]
    mx = jnp.max(logits, axis=1, keepdims=True)
    e = jnp.exp(logits - mx)
    probs = e / jnp.sum(e, axis=1, keepdims=True)
    pmx = jnp.max(probs, axis=1, keepdims=True)
    lane = jax.lax.broadcasted_iota(jnp.int32, (B, N_CLS), 1)
    cand = jnp.where(probs == pmx, lane, N_CLS)
    amin = jnp.min(cand, axis=1, keepdims=True)
    onehot = (lane == amin).astype(jnp.float32)
    allr = jax.lax.dot_general(
        f, wr_ref[...], dimension_numbers=(((1,), (0,)), ((), ())),
        preferred_element_type=jnp.float32) + br_ref[0:1, :]
    ei = jax.lax.broadcasted_iota(jnp.int32, (N_CLS, N_CLS * R_OUT), 0)
    ki = jax.lax.broadcasted_iota(jnp.int32, (N_CLS, N_CLS * R_OUT), 1)
    rep = (ki // R_OUT == ei).astype(jnp.float32)
    mask = jax.lax.dot_general(
        onehot, rep, dimension_numbers=(((1,), (0,)), ((), ())),
        preferred_element_type=jnp.float32)
    sel = allr * mask
    routed = (sel[:, 0:8] + sel[:, 8:16] + sel[:, 16:24] + sel[:, 24:32]
              + sel[:, 32:40] + sel[:, 40:48] + sel[:, 48:56] + sel[:, 56:64])
    p_ref[...] = probs
    r_ref[...] = routed


@functools.partial(jax.jit, static_argnums=())
def kernel(inputs, W_conv, b_conv, W_cls, b_cls, W_reg, b_reg):
    f32 = jnp.float32
    xp = jnp.pad(inputs, ((0, 0), (1, 1), (1, 7), (0, 0)))
    x2 = jnp.concatenate(
        [xp[:, :, 0:228, :].reshape(B, ROWS, QS, 12),
         xp[:, :, 4:232, :].reshape(B, ROWS, QS, 12)],
        axis=-1).reshape(B, XROWS, 24)
    u = jnp.arange(8)
    s = jnp.arange(4)
    dx = u[:, None] - s[None, :]
    valid = ((dx >= 0) & (dx <= 2)).astype(f32)
    wg = W_conv[:, jnp.clip(dx, 0, 2), :, :]
    wg = wg * valid[None, :, :, None, None]
    wg = wg.transpose(0, 1, 3, 2, 4).reshape(72, 256)
    whi = wg.astype(jnp.bfloat16)
    wlo = (wg - whi.astype(f32)).astype(jnp.bfloat16)
    w3 = jnp.concatenate([whi, wlo, whi], axis=0)
    b256 = jnp.tile(b_conv, 4).reshape(1, 256)
    wrf = W_reg.transpose(1, 0, 2).reshape(C_CONV, N_CLS * R_OUT)
    brf = b_reg.reshape(1, N_CLS * R_OUT)
    bcf = b_cls.reshape(1, N_CLS)

    feat256 = pl.pallas_call(
        _conv_gap_kernel,
        grid=(B,),
        in_specs=[
            pl.BlockSpec((1, XROWS, 24), lambda b: (b, 0, 0)),
            pl.BlockSpec((216, 256), lambda b: (0, 0)),
            pl.BlockSpec((1, 256), lambda b: (0, 0)),
        ],
        out_specs=pl.BlockSpec((1, 1, 256), lambda b: (b, 0, 0)),
        out_shape=jax.ShapeDtypeStruct((B, 1, 256), f32),
    )(x2, w3, b256)
    feat256 = feat256.reshape(B, 256)

    probs, routed = pl.pallas_call(
        _head_kernel,
        in_specs=[
            pl.BlockSpec(feat256.shape, lambda: (0, 0)),
            pl.BlockSpec((C_CONV, N_CLS), lambda: (0, 0)),
            pl.BlockSpec((1, N_CLS), lambda: (0, 0)),
            pl.BlockSpec((C_CONV, N_CLS * R_OUT), lambda: (0, 0)),
            pl.BlockSpec((1, N_CLS * R_OUT), lambda: (0, 0)),
        ],
        out_specs=[
            pl.BlockSpec((B, N_CLS), lambda: (0, 0)),
            pl.BlockSpec((B, R_OUT), lambda: (0, 0)),
        ],
        out_shape=[
            jax.ShapeDtypeStruct((B, N_CLS), f32),
            jax.ShapeDtypeStruct((B, R_OUT), f32),
        ],
    )(feat256, W_cls, bcf, wrf, brf)
    return (probs, routed)
